# Initial kernel scaffold; baseline (speedup 1.0000x reference)
import jax
import jax.numpy as jnp
from jax.experimental import pallas as pl


def kernel(x, time, context, len_x, len_context, len_time):
    raise NotImplementedError("write your pallas kernel here")


# speedup vs baseline: 1.3715x; 1.3715x over previous
import functools
import jax
import jax.numpy as jnp
from jax import lax
from jax.experimental import pallas as pl
from jax.experimental.pallas import tpu as pltpu
from jax.experimental.pallas import tpu_sc as plsc

_C = 5
_FILL = -999.0
_NC = 2
_NS = 16
_L = 16


def _make_sc_vals(B, Lx, Lc, Lt, D):
    n_workers = _NC * _NS
    assert n_workers % B == 0
    per_batch = n_workers // B
    chunk = Lc // per_batch
    assert chunk % _L == 0

    mesh = plsc.VectorSubcoreMesh(core_axis_name="c", subcore_axis_name="s",
                                  num_cores=_NC, num_subcores=_NS)

    @functools.partial(
        pl.kernel,
        out_type=jax.ShapeDtypeStruct((B, Lc), jnp.float32),
        mesh=mesh,
        scratch_types=[
            pltpu.VMEM((B,), jnp.int32),
            pltpu.VMEM((B,), jnp.int32),
            pltpu.VMEM((_L,), jnp.float32),
            pltpu.VMEM((Lt,), jnp.float32),
            pltpu.VMEM((chunk,), jnp.float32),
            pltpu.SemaphoreType.DMA,
        ],
    )
    def sc_vals(time2d_hbm, xf_hbm, lenx_hbm, lenc_hbm, vals_hbm,
                lenx_v, lenc_v, xg_v, row_v, out_v, sem):
        wid = lax.axis_index("s") * _NC + lax.axis_index("c")
        i = wid // per_batch
        j0 = (wid % per_batch) * chunk

        pltpu.sync_copy(lenx_hbm, lenx_v)
        pltpu.sync_copy(lenc_hbm, lenc_v)
        pltpu.sync_copy(time2d_hbm.at[i], row_v)
        lane = lax.iota(jnp.int32, _L)
        lx = lenx_v[pl.ds(i, 1)][0]
        lc = lenc_v[pl.ds(i, 1)][0]
        idx = jnp.where(lane < 8, i * (Lx * D) + _C, (i * Lx + lx - 1) * D + _C)
        pltpu.async_copy(xf_hbm.at[idx], xg_v, sem).wait()
        x0 = xg_v[pl.ds(0, 1)][0]
        xl = xg_v[pl.ds(8, 1)][0]
        t0 = row_v[pl.ds(0, 1)][0]
        tl = row_v[pl.ds(lx - 1, 1)][0]
        x0v = jnp.full((_L,), x0, jnp.float32)
        t0v = jnp.full((_L,), t0, jnp.float32)
        slope = (jnp.full((_L,), xl, jnp.float32) - x0v) / (
            jnp.full((_L,), tl, jnp.float32) - t0v)

        def step(it, carry):
            t = row_v[pl.ds(lx + j0 + it * _L, _L)]
            val = slope * (t - t0v) + x0v
            jv = j0 + it * _L + lane
            val = jnp.where(jv < lc, val, _FILL)
            out_v[pl.ds(it * _L, _L)] = val
            return carry

        lax.fori_loop(0, chunk // _L, step, 0)
        pltpu.sync_copy(out_v, vals_hbm.at[i, pl.ds(j0, chunk)])

    return sc_vals


def _tc_fill_body(vals_ref, out_ref):
    v = vals_ref[0]
    Lc, D = out_ref.shape[1], out_ref.shape[2]
    lane = lax.broadcasted_iota(jnp.int32, (Lc, D), 1)
    out_ref[0] = jnp.where(lane == _C, v, _FILL)


def kernel(x, time, context, len_x, len_context, len_time):
    B, Lx, D = x.shape
    Lc = context.shape[1]
    Lt = time.shape[1]

    sc_vals = _make_sc_vals(B, Lx, Lc, Lt, D)
    vals = sc_vals(time, x.reshape(-1),
                   len_x.astype(jnp.int32), len_context.astype(jnp.int32))

    out = pl.pallas_call(
        _tc_fill_body,
        grid=(B,),
        in_specs=[pl.BlockSpec((1, Lc, 1), lambda i: (i, 0, 0))],
        out_specs=pl.BlockSpec((1, Lc, D), lambda i: (i, 0, 0)),
        out_shape=jax.ShapeDtypeStruct((B, Lc, D), x.dtype),
    )(vals.reshape(B, Lc, 1))
    return out
